# baseline (speedup 1.0000x reference)
import functools

import jax
import jax.numpy as jnp
from jax import lax
from jax.experimental import pallas as pl
from jax.experimental.pallas import tpu as pltpu
from jax.experimental.pallas import tpu_sc as plsc


@functools.lru_cache(maxsize=None)
def _make_gather(batch: int, seq: int, dim: int):
    info = plsc.get_sparse_core_info()
    nc, ns = info.num_cores, info.num_subcores
    nw = nc * ns
    kb = batch // 128
    tgroups = nw // kb
    tspan = seq // tgroups
    assert batch % 128 == 0 and nw % kb == 0 and seq % tgroups == 0
    assert tspan % 2 == 0
    db = dim // 8
    assert dim % 8 == 0

    mesh = plsc.VectorSubcoreMesh(core_axis_name="c", subcore_axis_name="s")

    @functools.partial(
        pl.kernel,
        mesh=mesh,
        out_type=jax.ShapeDtypeStruct((seq, db, kb, 8, 128), jnp.float32),
        scratch_types=[
            pltpu.VMEM((tspan, 128), jnp.int32),
            pltpu.VMEM((2, 128, dim), jnp.float32),
            pltpu.VMEM((2, db, 8, 129), jnp.float32),
            pltpu.VMEM((16,), jnp.int32),
            pltpu.SemaphoreType.DMA,
            pltpu.SemaphoreType.DMA,
            pltpu.SemaphoreType.DMA,
            pltpu.SemaphoreType.DMA,
        ],
        compiler_params=pltpu.CompilerParams(
            use_tc_tiling_on_sc=False, needs_layout_passes=False),
    )
    def gather_kernel(text_hbm, aim_hbm, table_hbm, out_hbm,
                      idx_v, rows_v, slab_v, aim_v,
                      sg0, sg1, sw0, sw1):
        wid = lax.axis_index("s") * nc + lax.axis_index("c")
        k = wid % kb
        t0 = (wid // kb) * tspan
        pltpu.sync_copy(aim_hbm, aim_v)
        pltpu.sync_copy(
            text_hbm.at[pl.ds(t0, tspan), pl.ds(k * 128, 128)], idx_v)
        aim = aim_v[...][0]
        lane = lax.iota(jnp.int32, 16)

        @pl.when(aim < seq)
        def _mask():
            zeros = jnp.zeros((16,), jnp.int32)

            def mask_row(ti, carry):
                @pl.when(t0 + ti >= aim)
                def _():
                    for blk in range(8):
                        idx_v[ti, pl.ds(blk * 16, 16)] = zeros
                return carry

            lax.fori_loop(0, tspan, mask_row, 0)

        def fire_gather(ti, buf, sem):
            return pltpu.async_copy(
                table_hbm.at[idx_v.at[ti]], rows_v.at[buf], sem)

        nchunk = dim // 16
        dvecs = [lane + c * 16 for c in range(nchunk)]
        bbvecs = [v // 8 for v in dvecs]
        rvecs = [v % 8 for v in dvecs]

        def transpose(buf):
            rows = rows_v.at[buf]
            slab = slab_v.at[buf]

            @plsc.parallel_loop(0, 128, unroll=4)
            def brow(b):
                csplat = jnp.broadcast_to(b, (16,))
                for c in range(nchunk):
                    v = rows[b, pl.ds(c * 16, 16)]
                    plsc.store_scatter(
                        slab, [bbvecs[c], rvecs[c], csplat], v)

        def fire_write(ti, buf, sem):
            return pltpu.async_copy(
                slab_v.at[buf, :, :, pl.ds(0, 128)],
                out_hbm.at[t0 + ti, :, k], sem)

        def drain(copy):
            copy.wait()

        fire_gather(0, 0, sg0)

        def pair(p, carry):
            ti0 = 2 * p
            ti1 = 2 * p + 1
            fire_gather(ti1, 1, sg1)
            drain(pltpu.make_async_copy(
                table_hbm.at[idx_v.at[ti0]], rows_v.at[0], sg0))

            @pl.when(p > 0)
            def _():
                drain(pltpu.make_async_copy(
                    slab_v.at[0, :, :, pl.ds(0, 128)], out_hbm.at[t0 + ti0, :, k], sw0))
            transpose(0)
            fire_write(ti0, 0, sw0)

            @pl.when(p + 1 < tspan // 2)
            def _():
                fire_gather(ti0 + 2, 0, sg0)
            drain(pltpu.make_async_copy(
                table_hbm.at[idx_v.at[ti1]], rows_v.at[1], sg1))

            @pl.when(p > 0)
            def _():
                drain(pltpu.make_async_copy(
                    slab_v.at[1, :, :, pl.ds(0, 128)], out_hbm.at[t0 + ti1, :, k], sw1))
            transpose(1)
            fire_write(ti1, 1, sw1)
            return carry

        lax.fori_loop(0, tspan // 2, pair, 0)
        drain(pltpu.make_async_copy(
            slab_v.at[0, :, :, pl.ds(0, 128)], out_hbm.at[t0, :, k], sw0))
        drain(pltpu.make_async_copy(
            slab_v.at[1, :, :, pl.ds(0, 128)], out_hbm.at[t0, :, k], sw1))

    return gather_kernel


def kernel(text_bt, aim_seq_len, table):
    b, s = text_bt.shape
    dim = table.shape[1]
    aim_arr = jnp.broadcast_to(
        jnp.asarray(aim_seq_len, jnp.int32).reshape(1), (16,))
    out5 = _make_gather(b, s, dim)(text_bt.T, aim_arr, table)
    return out5.transpose(2, 4, 0, 1, 3).reshape(b, s, dim)


# speedup vs baseline: 1.0833x; 1.0833x over previous
import functools

import jax
import jax.numpy as jnp
from jax import lax
from jax.experimental import pallas as pl
from jax.experimental.pallas import tpu as pltpu
from jax.experimental.pallas import tpu_sc as plsc


@functools.lru_cache(maxsize=None)
def _make_gather(batch: int, seq: int, dim: int):
    info = plsc.get_sparse_core_info()
    nc, ns = info.num_cores, info.num_subcores
    nw = nc * ns
    kb = batch // 128
    tgroups = nw // kb
    tspan = seq // tgroups
    assert batch % 128 == 0 and nw % kb == 0 and seq % tgroups == 0
    assert tspan % 2 == 0
    db = dim // 8
    assert dim % 8 == 0

    mesh = plsc.VectorSubcoreMesh(core_axis_name="c", subcore_axis_name="s")

    @functools.partial(
        pl.kernel,
        mesh=mesh,
        out_type=jax.ShapeDtypeStruct((seq, db, kb, 8, 128), jnp.float32),
        scratch_types=[
            pltpu.VMEM((tspan, 128), jnp.int32),
            pltpu.VMEM((4, 128, dim), jnp.float32),
            pltpu.VMEM((2, db, 8, 129), jnp.float32),
            pltpu.VMEM((16,), jnp.int32),
            pltpu.SemaphoreType.DMA,
            pltpu.SemaphoreType.DMA,
            pltpu.SemaphoreType.DMA,
            pltpu.SemaphoreType.DMA,
            pltpu.SemaphoreType.DMA,
            pltpu.SemaphoreType.DMA,
        ],
        compiler_params=pltpu.CompilerParams(
            use_tc_tiling_on_sc=False, needs_layout_passes=False),
    )
    def gather_kernel(text_hbm, aim_hbm, table_hbm, out_hbm,
                      idx_v, rows_v, slab_v, aim_v,
                      sg0, sg1, sg2, sg3, sw0, sw1):
        wid = lax.axis_index("s") * nc + lax.axis_index("c")
        k = wid % kb
        t0 = (wid // kb) * tspan
        pltpu.sync_copy(aim_hbm, aim_v)
        pltpu.sync_copy(
            text_hbm.at[pl.ds(t0, tspan), pl.ds(k * 128, 128)], idx_v)
        aim = aim_v[...][0]
        lane = lax.iota(jnp.int32, 16)

        @pl.when(aim < seq)
        def _mask():
            zeros = jnp.zeros((16,), jnp.int32)

            def mask_row(ti, carry):
                @pl.when(t0 + ti >= aim)
                def _():
                    for blk in range(8):
                        idx_v[ti, pl.ds(blk * 16, 16)] = zeros
                return carry

            lax.fori_loop(0, tspan, mask_row, 0)

        def fire_gather(ti, buf, sem):
            return pltpu.async_copy(
                table_hbm.at[idx_v.at[ti]], rows_v.at[buf], sem)

        nchunk = dim // 16
        dvecs = [lane + c * 16 for c in range(nchunk)]
        bbvecs = [v // 8 for v in dvecs]
        rvecs = [v % 8 for v in dvecs]

        def transpose(rbuf, sbuf):
            rows = rows_v.at[rbuf]
            slab = slab_v.at[sbuf]

            @plsc.parallel_loop(0, 128, unroll=4)
            def brow(b):
                csplat = jnp.broadcast_to(b, (16,))
                for c in range(nchunk):
                    v = rows[b, pl.ds(c * 16, 16)]
                    plsc.store_scatter(
                        slab, [bbvecs[c], rvecs[c], csplat], v)

        def fire_write(ti, buf, sem):
            return pltpu.async_copy(
                slab_v.at[buf, :, :, pl.ds(0, 128)],
                out_hbm.at[t0 + ti, :, k], sem)

        def drain(copy):
            copy.wait()

        sgs = (sg0, sg1, sg2, sg3)
        sws = (sw0, sw1)
        nquads = tspan // 4
        tail = tspan - nquads * 4

        def drain_gather(ti, buf, sem):
            drain(pltpu.make_async_copy(
                table_hbm.at[idx_v.at[ti]], rows_v.at[buf], sem))

        def drain_write(ti, sbuf, sem):
            drain(pltpu.make_async_copy(
                slab_v.at[sbuf, :, :, pl.ds(0, 128)],
                out_hbm.at[t0 + ti, :, k], sem))

        for j in range(4):
            fire_gather(j, j, sgs[j])

        def quad(q, carry):
            for j in range(4):
                ti = 4 * q + j
                drain_gather(ti, j, sgs[j])

                @pl.when(ti >= 2)
                def _():
                    drain_write(ti - 2, j % 2, sws[j % 2])
                transpose(j, j % 2)

                @pl.when(4 * q + j + 4 < tspan)
                def _():
                    fire_gather(4 * q + j + 4, j, sgs[j])
                fire_write(ti, j % 2, sws[j % 2])
            return carry

        lax.fori_loop(0, nquads, quad, 0)
        for j in range(tail):
            ti = nquads * 4 + j
            drain_gather(ti, j, sgs[j])
            drain_write(ti - 2, j % 2, sws[j % 2])
            transpose(j, j % 2)
            fire_write(ti, j % 2, sws[j % 2])
        drain_write(tspan - 2, 0, sws[0])
        drain_write(tspan - 1, 1, sws[1])

    return gather_kernel


def kernel(text_bt, aim_seq_len, table):
    b, s = text_bt.shape
    dim = table.shape[1]
    aim_arr = jnp.broadcast_to(
        jnp.asarray(aim_seq_len, jnp.int32).reshape(1), (16,))
    out5 = _make_gather(b, s, dim)(text_bt.T, aim_arr, table)
    return out5.transpose(2, 4, 0, 1, 3).reshape(b, s, dim)
